# baseline (speedup 1.0000x reference)
import jax
import jax.numpy as jnp
from jax import lax
from jax.experimental import pallas as pl
from jax.experimental.pallas import tpu as pltpu
from jax.experimental.pallas import tpu_sc as plsc

NC = 2
NS = 16
NW = NC * NS
LANES = 16


def _make_sc_comb(N, H, T):
    PPW = N // NW
    JV = H // LANES
    mesh = plsc.VectorSubcoreMesh(core_axis_name="c", subcore_axis_name="s",
                                  num_cores=NC, num_subcores=NS)

    def body(t_hbm, s_hbm, tidx_hbm, out_hbm, idx_v, table_v, trow_v, comb_v):
        wid = lax.axis_index("s") * NC + lax.axis_index("c")
        p0 = wid * PPW

        pltpu.sync_copy(s_hbm.at[pl.ds(p0, PPW)], comb_v)
        pltpu.sync_copy(t_hbm, table_v)
        pltpu.sync_copy(tidx_hbm, idx_v)
        idxvec = idx_v[...]
        for j in range(JV):
            lane = jnp.arange(LANES, dtype=jnp.int32) + (j * LANES)
            trow_v[pl.ds(j * LANES, LANES)] = plsc.load_gather(
                table_v, [idxvec, lane])

        def row_add(p, _):
            for j in range(JV):
                sl = pl.ds(j * LANES, LANES)
                comb_v[p, sl] = comb_v[p, sl] + trow_v[sl]
            return 0
        lax.fori_loop(0, PPW, row_add, 0)
        pltpu.sync_copy(comb_v, out_hbm.at[pl.ds(p0, PPW)])

    return pl.kernel(
        body,
        out_type=jax.ShapeDtypeStruct((N, H), jnp.float32),
        mesh=mesh,
        compiler_params=pltpu.CompilerParams(use_tc_tiling_on_sc=False,
                                             needs_layout_passes=False),
        scratch_types=[
            pltpu.VMEM((LANES,), jnp.int32),
            pltpu.VMEM((T, H), jnp.float32),
            pltpu.VMEM((H,), jnp.float32),
            pltpu.VMEM((PPW, H), jnp.float32),
        ],
    )


def _tc_body(x_ref, comb_ref, out_ref):
    out_ref[...] = x_ref[...] + comb_ref[...]


def _tc_add(x, comb, bb):
    B, N, H = x.shape
    return pl.pallas_call(
        _tc_body,
        grid=(B // bb,),
        in_specs=[
            pl.BlockSpec((bb, N, H), lambda i: (i, 0, 0)),
            pl.BlockSpec((1, N, H), lambda i: (0, 0, 0)),
        ],
        out_specs=pl.BlockSpec((bb, N, H), lambda i: (i, 0, 0)),
        out_shape=jax.ShapeDtypeStruct((B, N, H), jnp.float32),
    )(x, comb[None])


def kernel(x, temporal_embed, spatial_embed, temporal_idx, num_patches):
    N, H = spatial_embed.shape
    T = temporal_embed.shape[0]
    tidx = jnp.full((LANES,), temporal_idx, dtype=jnp.int32)
    comb = _make_sc_comb(N, H, T)(temporal_embed, spatial_embed, tidx)
    return _tc_add(x, comb, 2)


# speedup vs baseline: 1.0240x; 1.0240x over previous
import jax
import jax.numpy as jnp
from jax import lax
from jax.experimental import pallas as pl
from jax.experimental.pallas import tpu as pltpu
from jax.experimental.pallas import tpu_sc as plsc

NC = 2
NS = 16
NW = NC * NS
LANES = 16


def _make_sc_comb(N, H, T):
    PPW = N // NW
    JV = H // LANES
    mesh = plsc.VectorSubcoreMesh(core_axis_name="c", subcore_axis_name="s",
                                  num_cores=NC, num_subcores=NS)

    def body(t_hbm, s_hbm, tidx_hbm, out_hbm, idx_v, table_v, trow_v, comb_v):
        wid = lax.axis_index("s") * NC + lax.axis_index("c")
        p0 = wid * PPW

        pltpu.sync_copy(s_hbm.at[pl.ds(p0, PPW)], comb_v)
        pltpu.sync_copy(t_hbm, table_v)
        pltpu.sync_copy(tidx_hbm, idx_v)
        idxvec = idx_v[...]
        for j in range(JV):
            lane = jnp.arange(LANES, dtype=jnp.int32) + (j * LANES)
            trow_v[pl.ds(j * LANES, LANES)] = plsc.load_gather(
                table_v, [idxvec, lane])

        def row_add(p, _):
            for j in range(JV):
                sl = pl.ds(j * LANES, LANES)
                comb_v[p, sl] = comb_v[p, sl] + trow_v[sl]
            return 0
        lax.fori_loop(0, PPW, row_add, 0)
        pltpu.sync_copy(comb_v, out_hbm.at[pl.ds(p0, PPW)])

    return pl.kernel(
        body,
        out_type=jax.ShapeDtypeStruct((N, H), jnp.float32),
        mesh=mesh,
        compiler_params=pltpu.CompilerParams(use_tc_tiling_on_sc=False,
                                             needs_layout_passes=False),
        scratch_types=[
            pltpu.VMEM((LANES,), jnp.int32),
            pltpu.VMEM((T, H), jnp.float32),
            pltpu.VMEM((H,), jnp.float32),
            pltpu.VMEM((PPW, H), jnp.float32),
        ],
    )


def _tc_body(x_ref, comb_ref, out_ref):
    out_ref[...] = x_ref[...] + comb_ref[...]


def _tc_add(x, comb, bb):
    B, N, H = x.shape
    return pl.pallas_call(
        _tc_body,
        grid=(B // bb,),
        in_specs=[
            pl.BlockSpec((bb, N, H), lambda i: (i, 0, 0)),
            pl.BlockSpec((1, N, H), lambda i: (0, 0, 0)),
        ],
        out_specs=pl.BlockSpec((bb, N, H), lambda i: (i, 0, 0)),
        out_shape=jax.ShapeDtypeStruct((B, N, H), jnp.float32),
        compiler_params=pltpu.CompilerParams(vmem_limit_bytes=100 * 1024 * 1024),
    )(x, comb[None])


def kernel(x, temporal_embed, spatial_embed, temporal_idx, num_patches):
    N, H = spatial_embed.shape
    T = temporal_embed.shape[0]
    tidx = jnp.full((LANES,), temporal_idx, dtype=jnp.int32)
    comb = _make_sc_comb(N, H, T)(temporal_embed, spatial_embed, tidx)
    return _tc_add(x, comb, 8)


# speedup vs baseline: 1.2375x; 1.2085x over previous
import jax
import jax.numpy as jnp
from jax import lax
from jax.experimental import pallas as pl
from jax.experimental.pallas import tpu as pltpu
from jax.experimental.pallas import tpu_sc as plsc

NC = 2
NS = 16
NW = NC * NS
LANES = 16


def _make_sc_comb(N, H, T):
    PPW = N // NW
    JV = H // LANES
    mesh = plsc.VectorSubcoreMesh(core_axis_name="c", subcore_axis_name="s",
                                  num_cores=NC, num_subcores=NS)

    def body(t_hbm, s_hbm, tidx_hbm, out_hbm, idx_v, table_v, trow_v, comb_v):
        wid = lax.axis_index("s") * NC + lax.axis_index("c")
        p0 = wid * PPW

        pltpu.sync_copy(s_hbm.at[pl.ds(p0, PPW)], comb_v)
        pltpu.sync_copy(t_hbm, table_v)
        pltpu.sync_copy(tidx_hbm, idx_v)
        idxvec = idx_v[...]
        for j in range(JV):
            lane = jnp.arange(LANES, dtype=jnp.int32) + (j * LANES)
            trow_v[pl.ds(j * LANES, LANES)] = plsc.load_gather(
                table_v, [idxvec, lane])

        def row_add(p, _):
            for j in range(JV):
                sl = pl.ds(j * LANES, LANES)
                comb_v[p, sl] = comb_v[p, sl] + trow_v[sl]
            return 0
        lax.fori_loop(0, PPW, row_add, 0)
        pltpu.sync_copy(comb_v, out_hbm.at[pl.ds(p0, PPW)])

    return pl.kernel(
        body,
        out_type=jax.ShapeDtypeStruct((N, H), jnp.float32),
        mesh=mesh,
        compiler_params=pltpu.CompilerParams(use_tc_tiling_on_sc=False,
                                             needs_layout_passes=False),
        scratch_types=[
            pltpu.VMEM((LANES,), jnp.int32),
            pltpu.VMEM((T, H), jnp.float32),
            pltpu.VMEM((H,), jnp.float32),
            pltpu.VMEM((PPW, H), jnp.float32),
        ],
    )


def _tc_body(x_ref, comb_ref, out_ref):
    out_ref[...] = x_ref[...] + comb_ref[...]


def _tc_add(x, comb, bb):
    B, N, H = x.shape
    return pl.pallas_call(
        _tc_body,
        grid=(B // bb,),
        in_specs=[
            pl.BlockSpec((bb, N, H), lambda i: (i, 0, 0)),
            pl.BlockSpec((1, N, H), lambda i: (0, 0, 0)),
        ],
        out_specs=pl.BlockSpec((bb, N, H), lambda i: (i, 0, 0)),
        out_shape=jax.ShapeDtypeStruct((B, N, H), jnp.float32),
        compiler_params=pltpu.CompilerParams(vmem_limit_bytes=100 * 1024 * 1024),
    )(x, comb[None])


def kernel(x, temporal_embed, spatial_embed, temporal_idx, num_patches):
    N, H = spatial_embed.shape
    T = temporal_embed.shape[0]
    comb = spatial_embed + temporal_embed[temporal_idx][None]
    return _tc_add(x, comb, 8)
